# baseline (speedup 1.0000x reference)
import functools

import jax
import jax.numpy as jnp
from jax import lax
from jax.experimental import pallas as pl
from jax.experimental.pallas import tpu as pltpu
from jax.experimental.pallas import tpu_sc as plsc

_NB = 32
_BATCH = 16384
_NC = 2
_NS = 16
_NW = _NC * _NS
_G = 128
_S2 = 8


def _mesh():
  return plsc.VectorSubcoreMesh(
      core_axis_name="c", subcore_axis_name="s",
      num_cores=_NC, num_subcores=_NS)


def _ring_pipeline(ng, S, fire_gathers, wait_gathers, fire_writes,
                   wait_writes):
  L = S // 2
  if ng <= S:
    for g in range(ng):
      fire_gathers(g, g)
    for g in range(ng):
      wait_gathers(g, g)
      fire_writes(g, g)
    for g in range(ng):
      wait_writes(g, g)
    return

  for b in range(L):
    fire_gathers(b, b)
  for b in range(S):
    g = b
    slot_n = (b + L) % S
    if g + L >= S:
      wait_writes(g - L, slot_n)
    fire_gathers(g + L, slot_n)
    wait_gathers(g, b)
    fire_writes(g, b)

  def outer(t, carry):
    for b in range(S):
      g = t * S + b
      slot_n = (b + L) % S
      wait_writes(g - L, slot_n)
      fire_gathers(g + L, slot_n)
      wait_gathers(g, b)
      fire_writes(g, b)
    return carry

  lax.fori_loop(1, ng // S - 1, outer, 0)

  t_last = ng // S - 1
  for b in range(S):
    g = t_last * S + b
    slot_n = (b + L) % S
    if g + L < ng:
      wait_writes(g - L, slot_n)
      fire_gathers(g + L, slot_n)
    wait_gathers(g, b)
    fire_writes(g, b)
  for b in range(S):
    wait_writes(ng - S + b, b)


def _hop2_body(n_per_w, idx_hbm, adj_hbm, out, idx_v, buf, *sems):
  gsems, wsems = sems[:_S2], sems[_S2:]
  idx2d_v = None
  if len(idx_hbm.shape) == 2:
    idx_v, idx2d_v = idx_v
  wid = lax.axis_index("s") * _NC + lax.axis_index("c")
  base = wid * n_per_w
  if idx2d_v is None:
    pltpu.sync_copy(idx_hbm.at[pl.ds(base, n_per_w)], idx_v)
  else:
    rows = n_per_w // _NB
    pltpu.sync_copy(idx_hbm.at[pl.ds(wid * rows, rows)], idx2d_v)

    def flat_body(r, carry):
      row = idx2d_v.at[r]
      idx_v[pl.ds(r * _NB, 16)] = row[pl.ds(0, 16)]
      idx_v[pl.ds(r * _NB + 16, 16)] = row[pl.ds(16, 16)]
      return carry

    lax.fori_loop(0, rows, flat_body, 0)

  def fire_gathers(g, slot):
    off = idx_v.at[pl.ds(g * _G, _G)]
    pltpu.async_copy(adj_hbm.at[off], buf.at[slot], gsems[slot])

  def wait_gathers(g, slot):
    off = idx_v.at[pl.ds(g * _G, _G)]
    pltpu.make_async_copy(adj_hbm.at[off], buf.at[slot], gsems[slot]).wait()

  def out_slice(g):
    return out.at[pl.ds(base + g * _G, _G)]

  def fire_writes(g, slot):
    pltpu.async_copy(buf.at[slot], out_slice(g), wsems[slot])

  def wait_writes(g, slot):
    pltpu.make_async_copy(buf.at[slot], out_slice(g), wsems[slot]).wait()

  _ring_pipeline(n_per_w // _G, _S2, fire_gathers, wait_gathers,
                 fire_writes, wait_writes)


_PARAMS = pltpu.CompilerParams(use_tc_tiling_on_sc=False)


def _hop2(idx, adj):
  n = idx.size
  n_per_w = n // _NW
  body = functools.partial(_hop2_body, n_per_w)
  out_type = jax.ShapeDtypeStruct((n, _NB), jnp.int32)
  if idx.ndim == 2:
    idx_scratch = [pltpu.VMEM((n_per_w,), jnp.int32),
                   pltpu.VMEM((n_per_w // _NB, _NB), jnp.int32)]
  else:
    idx_scratch = pltpu.VMEM((n_per_w,), jnp.int32)
  scratch = [
      idx_scratch,
      pltpu.VMEM((_S2, _G, _NB), jnp.int32),
  ] + [pltpu.SemaphoreType.DMA] * (2 * _S2)
  return pl.kernel(
      body, out_type=out_type, mesh=_mesh(), scratch_types=scratch,
      compiler_params=_PARAMS)(idx, adj)


def kernel(entity, adj_entity, adj_relation):
  ent_flat = entity.reshape(-1)
  ent1 = _hop2(ent_flat, adj_entity)
  ent2 = _hop2(ent1, adj_entity)
  rel1 = _hop2(ent_flat, adj_relation)
  half = _BATCH // 2
  rel2a = _hop2(ent1[:half], adj_relation)
  rel2b = _hop2(ent1[half:], adj_relation)
  rel2 = jnp.concatenate([rel2a.reshape(half, _NB * _NB),
                          rel2b.reshape(half, _NB * _NB)], axis=0)
  return (entity,
          ent1,
          ent2.reshape(_BATCH, _NB * _NB),
          rel1,
          rel2)


# speedup vs baseline: 1.1529x; 1.1529x over previous
import functools

import jax
import jax.numpy as jnp
from jax import lax
from jax.experimental import pallas as pl
from jax.experimental.pallas import tpu as pltpu
from jax.experimental.pallas import tpu_sc as plsc

_NB = 32
_BATCH = 16384
_NC = 2
_NS = 16
_NW = _NC * _NS
_G = 128
_S2 = 8


def _mesh():
  return plsc.VectorSubcoreMesh(
      core_axis_name="c", subcore_axis_name="s",
      num_cores=_NC, num_subcores=_NS)


def _ring_pipeline(ng, S, fire_gathers, wait_gathers, fire_writes,
                   wait_writes):
  L = S // 2
  if ng <= S:
    for g in range(ng):
      fire_gathers(g, g)
    for g in range(ng):
      wait_gathers(g, g)
      fire_writes(g, g)
    for g in range(ng):
      wait_writes(g, g)
    return

  for b in range(L):
    fire_gathers(b, b)
  for b in range(S):
    g = b
    slot_n = (b + L) % S
    if g + L >= S:
      wait_writes(g - L, slot_n)
    fire_gathers(g + L, slot_n)
    wait_gathers(g, b)
    fire_writes(g, b)

  def outer(t, carry):
    for b in range(S):
      g = t * S + b
      slot_n = (b + L) % S
      wait_writes(g - L, slot_n)
      fire_gathers(g + L, slot_n)
      wait_gathers(g, b)
      fire_writes(g, b)
    return carry

  lax.fori_loop(1, ng // S - 1, outer, 0)

  t_last = ng // S - 1
  for b in range(S):
    g = t_last * S + b
    slot_n = (b + L) % S
    if g + L < ng:
      wait_writes(g - L, slot_n)
      fire_gathers(g + L, slot_n)
    wait_gathers(g, b)
    fire_writes(g, b)
  for b in range(S):
    wait_writes(ng - S + b, b)


def _hop2_body(n_per_w, idx_hbm, adj_hbm, out, idx_v, buf, *sems):
  gsems, wsems = sems[:_S2], sems[_S2:]
  idx2d_v = None
  if len(idx_hbm.shape) == 2:
    idx_v, idx2d_v = idx_v
  wid = lax.axis_index("s") * _NC + lax.axis_index("c")
  base = wid * n_per_w
  if idx2d_v is None:
    pltpu.sync_copy(idx_hbm.at[pl.ds(base, n_per_w)], idx_v)
  else:
    rows = n_per_w // _NB
    pltpu.sync_copy(idx_hbm.at[pl.ds(wid * rows, rows)], idx2d_v)

    def flat_body(r, carry):
      row = idx2d_v.at[r]
      idx_v[pl.ds(r * _NB, 16)] = row[pl.ds(0, 16)]
      idx_v[pl.ds(r * _NB + 16, 16)] = row[pl.ds(16, 16)]
      return carry

    lax.fori_loop(0, rows, flat_body, 0)

  def fire_gathers(g, slot):
    off = idx_v.at[pl.ds(g * _G, _G)]
    pltpu.async_copy(adj_hbm.at[off], buf.at[slot], gsems[slot])

  def wait_gathers(g, slot):
    off = idx_v.at[pl.ds(g * _G, _G)]
    pltpu.make_async_copy(adj_hbm.at[off], buf.at[slot], gsems[slot]).wait()

  def out_slice(g):
    return out.at[pl.ds(base + g * _G, _G)]

  def fire_writes(g, slot):
    pltpu.async_copy(buf.at[slot], out_slice(g), wsems[slot])

  def wait_writes(g, slot):
    pltpu.make_async_copy(buf.at[slot], out_slice(g), wsems[slot]).wait()

  _ring_pipeline(n_per_w // _G, _S2, fire_gathers, wait_gathers,
                 fire_writes, wait_writes)


_PARAMS = pltpu.CompilerParams(use_tc_tiling_on_sc=False)


def _hop2(idx, adj):
  n = idx.size
  n_per_w = n // _NW
  body = functools.partial(_hop2_body, n_per_w)
  out_type = jax.ShapeDtypeStruct((n, _NB), jnp.int32)
  if idx.ndim == 2:
    idx_scratch = [pltpu.VMEM((n_per_w,), jnp.int32),
                   pltpu.VMEM((n_per_w // _NB, _NB), jnp.int32)]
  else:
    idx_scratch = pltpu.VMEM((n_per_w,), jnp.int32)
  scratch = [
      idx_scratch,
      pltpu.VMEM((_S2, _G, _NB), jnp.int32),
  ] + [pltpu.SemaphoreType.DMA] * (2 * _S2)
  return pl.kernel(
      body, out_type=out_type, mesh=_mesh(), scratch_types=scratch,
      compiler_params=_PARAMS)(idx, adj)


def kernel(entity, adj_entity, adj_relation):
  ent_flat = entity.reshape(-1)
  ent1 = _hop2(ent_flat, adj_entity)
  ent2 = _hop2(ent1, adj_entity)
  rel1 = _hop2(ent_flat, adj_relation)
  rel2 = _hop2(ent1, adj_relation)
  return (entity,
          ent1,
          ent2.reshape(_BATCH, _NB * _NB),
          rel1,
          rel2.reshape(_BATCH, _NB * _NB))
